# baseline (speedup 1.0000x reference)
import functools

import jax
import jax.numpy as jnp
from jax import lax
from jax.experimental import pallas as pl
from jax.experimental.pallas import tpu as pltpu
from jax.experimental.pallas import tpu_sc as plsc

_NC, _NS, _L = 2, 16, 16
_NW = _NC * _NS


def _emb_gather(ids1d, emb_table):
    nrow = ids1d.shape[0]
    D = emb_table.shape[1]
    mesh = plsc.VectorSubcoreMesh(core_axis_name="c", subcore_axis_name="s")

    @functools.partial(
        pl.kernel, mesh=mesh,
        out_type=jax.ShapeDtypeStruct((nrow, D), jnp.float32),
        scratch_types=[
            pltpu.VMEM((256,), jnp.int32),
            pltpu.VMEM((4, 32, D), jnp.float32),
            pltpu.SemaphoreType.DMA,
            pltpu.SemaphoreType.DMA,
            pltpu.SemaphoreType.DMA,
            pltpu.SemaphoreType.DMA,
            pltpu.SemaphoreType.DMA,
            pltpu.SemaphoreType.DMA,
            pltpu.SemaphoreType.DMA,
            pltpu.SemaphoreType.DMA,
        ],
    )
    def k(ids_hbm, table_hbm, out_hbm, idx_v, bufs,
          sg0, sg1, sg2, sg3, sw0, sw1, sw2, sw3):
        wid = lax.axis_index("s") * _NC + lax.axis_index("c")
        pltpu.sync_copy(ids_hbm.at[pl.ds(wid * 256, 256)], idx_v)
        sg = [sg0, sg1, sg2, sg3]
        sw = [sw0, sw1, sw2, sw3]
        gcp = {}
        wcp = {}
        for b in range(4):
            gcp[b] = pltpu.async_copy(
                table_hbm.at[idx_v.at[pl.ds(b * 32, 32)]], bufs.at[b], sg[b])
        for c in range(4):
            gcp[c].wait()
            wcp[c] = pltpu.async_copy(
                bufs.at[c], out_hbm.at[pl.ds(wid * 256 + c * 32, 32)], sw[c])
        for c in range(4, 8):
            d = c - 4
            wcp[d].wait()
            gcp[c] = pltpu.async_copy(
                table_hbm.at[idx_v.at[pl.ds(c * 32, 32)]], bufs.at[d], sg[d])
        for c in range(4, 8):
            d = c - 4
            gcp[c].wait()
            wcp[c] = pltpu.async_copy(
                bufs.at[d], out_hbm.at[pl.ds(wid * 256 + c * 32, 32)], sw[d])
        for c in range(4, 8):
            wcp[c].wait()

    return k(ids1d, emb_table)


def _pool_and_winner(spans1d, E):
    mesh = plsc.VectorSubcoreMesh(core_axis_name="c", subcore_axis_name="s")
    nspan = spans1d.shape[0] // 16
    nslot = E.shape[0]

    @functools.partial(
        pl.kernel, mesh=mesh,
        out_type=(jax.ShapeDtypeStruct((nspan, 256), jnp.float32),
                  jax.ShapeDtypeStruct((2, nslot), jnp.int32)),
        compiler_params=pltpu.CompilerParams(needs_layout_passes=False),
        scratch_types=[
            pltpu.VMEM((8192,), jnp.int32),
            pltpu.VMEM((4, 64, 256), jnp.float32),
            pltpu.VMEM((4, 4, 256), jnp.float32),
            pltpu.VMEM((nslot,), jnp.int32),
            pltpu.VMEM((16,), jnp.int32),
            pltpu.VMEM((512,), jnp.int32),
            pltpu.VMEM((512,), jnp.int32),
            pltpu.VMEM_SHARED((16, nslot), jnp.int32),
            pltpu.SemaphoreType.DMA,
            pltpu.SemaphoreType.DMA,
            pltpu.SemaphoreType.DMA,
            pltpu.SemaphoreType.DMA,
            pltpu.SemaphoreType.DMA,
            pltpu.SemaphoreType.DMA,
            pltpu.SemaphoreType.DMA,
            pltpu.SemaphoreType.DMA,
        ],
    )
    def k(spans_hbm, e_hbm, pooled_hbm, wsc_hbm,
          idx_v, rows_v, pool_v, wloc, tscr, wacc, wtmp, shared,
          sg0, sg1, sg2, sg3, sp0, sp1, sp2, sp3):
        cid = lax.axis_index("c")
        sid = lax.axis_index("s")
        wid = sid * _NC + cid
        pltpu.sync_copy(spans_hbm.at[pl.ds(wid * 8192, 8192)], idx_v)

        def init_body(i, _):
            wloc[pl.ds(i * 16, 16)] = jnp.full((16,), -1, jnp.int32)
            return 0
        lax.fori_loop(0, nslot // 16, init_body, 0)

        semg = [sg0, sg1, sg2, sg3]
        semp = [sp0, sp1, sp2, sp3]
        inv = jnp.float32(1.0 / 16.0)

        def fire(c, d):
            return pltpu.async_copy(
                e_hbm.at[idx_v.at[pl.ds(c * 64, 64)]], rows_v.at[d],
                semg[d])

        def wait_gather(d):
            pltpu.make_async_copy(
                e_hbm.at[pl.ds(0, 64)], rows_v.at[d], semg[d]).wait()

        def wait_pwrite(d):
            pltpu.make_async_copy(
                pool_v.at[d], pooled_hbm.at[pl.ds(0, 4)], semp[d]).wait()

        def reduce_chunk(d):
            for s in range(4):
                def rbody(r2, accs):
                    r = s * 16 + r2 * 2
                    out = []
                    for cc in range(16):
                        a = accs[cc] + rows_v[d, r, pl.ds(cc * 16, 16)]
                        out.append(a + rows_v[d, r + 1, pl.ds(cc * 16, 16)])
                    return tuple(out)
                accs = lax.fori_loop(
                    0, 8, rbody,
                    tuple(jnp.zeros((16,), jnp.float32) for _ in range(16)))
                for cc in range(16):
                    pool_v[d, s, pl.ds(cc * 16, 16)] = accs[cc] * inv

        def pwrite(c, d):
            return pltpu.async_copy(
                pool_v.at[d],
                pooled_hbm.at[pl.ds(wid * 512 + c * 4, 4)], semp[d])

        for b in range(4):
            fire(b, b)
        for d in range(4):
            wait_gather(d)
            reduce_chunk(d)
            fire(d + 4, d)
            pwrite(d, d)

        def gbody(g, _):
            for d in range(4):
                c = g * 4 + d
                wait_gather(d)
                wait_pwrite(d)
                reduce_chunk(d)
                fire(c + 4, d)
                pwrite(c, d)
            return 0
        lax.fori_loop(1, 31, gbody, 0)

        for d in range(4):
            wait_gather(d)
            wait_pwrite(d)
            reduce_chunk(d)
            pwrite(124 + d, d)
        for d in range(4):
            wait_pwrite(d)

        lane = lax.broadcasted_iota(jnp.int32, (16,), 0)

        def win_body(g, _):
            t0 = g * 16 + lane
            m = jnp.full((16,), -1, jnp.int32)

            def maxj(j, m):
                v = plsc.load_gather(idx_v, [t0 * 16 + j])
                return jnp.maximum(m, v)
            m = lax.fori_loop(0, 16, maxj, m)
            tgtv = m + 1
            iv = wid * 512 + t0
            key = (tgtv << 14) | iv
            skey, _sval = plsc.sort_key_val(key, key)
            tgt_s = skey >> 14
            i_s = skey & 16383
            tscr[...] = tgt_s
            nxt = plsc.load_gather(tscr, [jnp.minimum(lane + 1, 15)])
            is_last = (tgt_s != nxt) | (lane == 15)
            plsc.store_scatter(wloc, [tgt_s], i_s, mask=is_last)
            return 0
        lax.fori_loop(0, 32, win_body, 0)

        pltpu.sync_copy(wloc, shared.at[sid])
        plsc.subcore_barrier()
        pltpu.sync_copy(shared.at[0, pl.ds(sid * 512, 512)], wacc)

        def tile_body(t, _):
            pltpu.sync_copy(shared.at[t, pl.ds(sid * 512, 512)], wtmp)

            def max_body(kk, _):
                wacc[pl.ds(kk * 16, 16)] = jnp.maximum(
                    wacc[pl.ds(kk * 16, 16)], wtmp[pl.ds(kk * 16, 16)])
                return 0
            lax.fori_loop(0, 32, max_body, 0)
            return 0
        lax.fori_loop(1, 16, tile_body, 0)
        pltpu.sync_copy(wacc, wsc_hbm.at[cid, pl.ds(sid * 512, 512)])

    return k(spans1d, E)


def _merge(E, C, wsc):
    nslot = E.shape[0]
    mesh = plsc.VectorSubcoreMesh(core_axis_name="c", subcore_axis_name="s")

    @functools.partial(
        pl.kernel, mesh=mesh,
        out_type=jax.ShapeDtypeStruct((nslot, 256), jnp.float32),
        compiler_params=pltpu.CompilerParams(needs_layout_passes=False),
        scratch_types=[
            pltpu.VMEM((256,), jnp.int32),
            pltpu.VMEM((256,), jnp.int32),
            pltpu.VMEM((256,), jnp.int32),
            pltpu.VMEM((256,), jnp.int32),
            pltpu.VMEM((128, 256), jnp.float32),
            pltpu.VMEM((128, 256), jnp.float32),
            pltpu.SemaphoreType.DMA,
            pltpu.SemaphoreType.DMA,
        ],
    )
    def k(e_hbm, c_hbm, wsc_hbm, out_hbm, w0, w1, wmax, cidx, crows, erows,
          semc, seme):
        wid = lax.axis_index("s") * _NC + lax.axis_index("c")
        base = wid * 256
        pltpu.sync_copy(wsc_hbm.at[0, pl.ds(base, 256)], w0)
        pltpu.sync_copy(wsc_hbm.at[1, pl.ds(base, 256)], w1)

        def max_body(kk, _):
            sl = pl.ds(kk * 16, 16)
            m = jnp.maximum(w0[sl], w1[sl])
            wmax[sl] = m
            cidx[sl] = jnp.maximum(m, 0)
            return 0
        lax.fori_loop(0, 16, max_body, 0)

        zeros16 = jnp.zeros((16,), jnp.int32)
        for h in range(2):
            g1 = pltpu.async_copy(
                c_hbm.at[cidx.at[pl.ds(h * 128, 128)]], crows, semc)
            g2 = pltpu.async_copy(
                e_hbm.at[pl.ds(base + h * 128, 128)], erows, seme)
            g1.wait()
            g2.wait()

            def sel_body(r, _):
                m = plsc.load_gather(wmax, [zeros16 + (h * 128 + r)]) >= 0
                for cc in range(16):
                    sl = pl.ds(cc * 16, 16)
                    erows[r, sl] = jnp.where(m, crows[r, sl], erows[r, sl])
                return 0
            lax.fori_loop(0, 128, sel_body, 0)
            pltpu.sync_copy(erows, out_hbm.at[pl.ds(base + h * 128, 128)])

    return k(E, C, wsc)


def _down_body(g_ref, w_ref, b_ref, o_ref):
    o_ref[...] = (
        jnp.dot(g_ref[...], w_ref[...], preferred_element_type=jnp.float32)
        + b_ref[...]
    )


def _down_matmul(g, w, b):
    m, k = g.shape
    n = w.shape[1]
    bm = 512
    return pl.pallas_call(
        _down_body,
        grid=(m // bm,),
        in_specs=[
            pl.BlockSpec((bm, k), lambda i: (i, 0)),
            pl.BlockSpec((k, n), lambda i: (0, 0)),
            pl.BlockSpec((n,), lambda i: (0,)),
        ],
        out_specs=pl.BlockSpec((bm, n), lambda i: (i, 0)),
        out_shape=jax.ShapeDtypeStruct((m, n), jnp.float32),
    )(g, w, b)


def _comp_body(p_ref, w_ref, b_ref, o_ref):
    o_ref[...] = jnp.tanh(
        jnp.dot(p_ref[...], w_ref[...], preferred_element_type=jnp.float32)
        + b_ref[...])


def _comp_matmul(p, w, b):
    m, k = p.shape
    n = w.shape[1]
    bm = 1024
    return pl.pallas_call(
        _comp_body,
        grid=(m // bm,),
        in_specs=[
            pl.BlockSpec((bm, k), lambda i: (i, 0)),
            pl.BlockSpec((k, n), lambda i: (0, 0)),
            pl.BlockSpec((n,), lambda i: (0,)),
        ],
        out_specs=pl.BlockSpec((bm, n), lambda i: (i, 0)),
        out_shape=jax.ShapeDtypeStruct((m, n), jnp.float32),
    )(p, w, b)


def kernel(chunk_input_ids, chunk_tgt_ids, merged_spans,
           opening_nt_start_idx, closing_nt_start_idx,
           emb_table, W_down, b_down, W_comp, b_comp):
    bsz, seqlen = chunk_input_ids.shape
    ids1d = chunk_input_ids.reshape(-1)
    G = _emb_gather(ids1d, emb_table)
    E = _down_matmul(G, W_down, b_down)

    spans1d = merged_spans.reshape(-1)
    pooled, wsc = _pool_and_winner(spans1d, E)
    C = _comp_matmul(pooled, W_comp, b_comp)
    out = _merge(E, C, wsc)
    return out.reshape(bsz, seqlen, -1)


# speedup vs baseline: 1.9738x; 1.9738x over previous
import functools

import jax
import jax.numpy as jnp
from jax import lax
from jax.experimental import pallas as pl
from jax.experimental.pallas import tpu as pltpu
from jax.experimental.pallas import tpu_sc as plsc

_NC, _NS, _L = 2, 16, 16
_NW = _NC * _NS


def _emb_gather(ids1d, emb_table):
    nrow = ids1d.shape[0]
    D = emb_table.shape[1]
    mesh = plsc.VectorSubcoreMesh(core_axis_name="c", subcore_axis_name="s")

    @functools.partial(
        pl.kernel, mesh=mesh,
        out_type=jax.ShapeDtypeStruct((nrow, D), jnp.float32),
        scratch_types=[
            pltpu.VMEM((256,), jnp.int32),
            pltpu.VMEM((4, 32, D), jnp.float32),
            pltpu.SemaphoreType.DMA,
            pltpu.SemaphoreType.DMA,
            pltpu.SemaphoreType.DMA,
            pltpu.SemaphoreType.DMA,
            pltpu.SemaphoreType.DMA,
            pltpu.SemaphoreType.DMA,
            pltpu.SemaphoreType.DMA,
            pltpu.SemaphoreType.DMA,
        ],
    )
    def k(ids_hbm, table_hbm, out_hbm, idx_v, bufs,
          sg0, sg1, sg2, sg3, sw0, sw1, sw2, sw3):
        wid = lax.axis_index("s") * _NC + lax.axis_index("c")
        pltpu.sync_copy(ids_hbm.at[pl.ds(wid * 256, 256)], idx_v)
        sg = [sg0, sg1, sg2, sg3]
        sw = [sw0, sw1, sw2, sw3]
        gcp = {}
        wcp = {}
        for b in range(4):
            gcp[b] = pltpu.async_copy(
                table_hbm.at[idx_v.at[pl.ds(b * 32, 32)]], bufs.at[b], sg[b])
        for c in range(4):
            gcp[c].wait()
            wcp[c] = pltpu.async_copy(
                bufs.at[c], out_hbm.at[pl.ds(wid * 256 + c * 32, 32)], sw[c])
        for c in range(4, 8):
            d = c - 4
            wcp[d].wait()
            gcp[c] = pltpu.async_copy(
                table_hbm.at[idx_v.at[pl.ds(c * 32, 32)]], bufs.at[d], sg[d])
        for c in range(4, 8):
            d = c - 4
            gcp[c].wait()
            wcp[c] = pltpu.async_copy(
                bufs.at[d], out_hbm.at[pl.ds(wid * 256 + c * 32, 32)], sw[d])
        for c in range(4, 8):
            wcp[c].wait()

    return k(ids1d, emb_table)


def _pool_and_winner(spans1d, E):
    mesh = plsc.VectorSubcoreMesh(core_axis_name="c", subcore_axis_name="s")
    nspan = spans1d.shape[0] // 16
    nslot = E.shape[0]

    @functools.partial(
        pl.kernel, mesh=mesh,
        out_type=(jax.ShapeDtypeStruct((nspan, 256), jnp.float32),
                  jax.ShapeDtypeStruct((2, nslot), jnp.int32)),
        compiler_params=pltpu.CompilerParams(needs_layout_passes=False),
        scratch_types=[
            pltpu.VMEM((8192,), jnp.int32),
            pltpu.VMEM((4, 64, 256), jnp.float32),
            pltpu.VMEM((4, 4, 256), jnp.float32),
            pltpu.VMEM((nslot,), jnp.int32),
            pltpu.VMEM((16,), jnp.int32),
            pltpu.VMEM((512,), jnp.int32),
            pltpu.VMEM((512,), jnp.int32),
            pltpu.VMEM_SHARED((16, nslot), jnp.int32),
            pltpu.SemaphoreType.DMA,
            pltpu.SemaphoreType.DMA,
            pltpu.SemaphoreType.DMA,
            pltpu.SemaphoreType.DMA,
            pltpu.SemaphoreType.DMA,
            pltpu.SemaphoreType.DMA,
            pltpu.SemaphoreType.DMA,
            pltpu.SemaphoreType.DMA,
        ],
    )
    def k(spans_hbm, e_hbm, pooled_hbm, wsc_hbm,
          idx_v, rows_v, pool_v, wloc, tscr, wacc, wtmp, shared,
          sg0, sg1, sg2, sg3, sp0, sp1, sp2, sp3):
        cid = lax.axis_index("c")
        sid = lax.axis_index("s")
        wid = sid * _NC + cid
        pltpu.sync_copy(spans_hbm.at[pl.ds(wid * 8192, 8192)], idx_v)

        def init_body(i, _):
            wloc[pl.ds(i * 16, 16)] = jnp.full((16,), -1, jnp.int32)
            return 0
        lax.fori_loop(0, nslot // 16, init_body, 0)

        semg = [sg0, sg1, sg2, sg3]
        semp = [sp0, sp1, sp2, sp3]
        inv = jnp.float32(1.0 / 16.0)

        def fire(c, d):
            return pltpu.async_copy(
                e_hbm.at[idx_v.at[pl.ds(c * 64, 64)]], rows_v.at[d],
                semg[d])

        def wait_gather(d):
            pltpu.make_async_copy(
                e_hbm.at[pl.ds(0, 64)], rows_v.at[d], semg[d]).wait()

        def wait_pwrite(d):
            pltpu.make_async_copy(
                pool_v.at[d], pooled_hbm.at[pl.ds(0, 4)], semp[d]).wait()

        def reduce_chunk(d):
            for s in range(4):
                def rbody(r2, accs):
                    r = s * 16 + r2 * 2
                    out = []
                    for cc in range(16):
                        a = accs[cc] + rows_v[d, r, pl.ds(cc * 16, 16)]
                        out.append(a + rows_v[d, r + 1, pl.ds(cc * 16, 16)])
                    return tuple(out)
                accs = lax.fori_loop(
                    0, 8, rbody,
                    tuple(jnp.zeros((16,), jnp.float32) for _ in range(16)))
                for cc in range(16):
                    pool_v[d, s, pl.ds(cc * 16, 16)] = accs[cc] * inv

        def pwrite(c, d):
            return pltpu.async_copy(
                pool_v.at[d],
                pooled_hbm.at[pl.ds(wid * 512 + c * 4, 4)], semp[d])

        for b in range(4):
            fire(b, b)
        for d in range(4):
            wait_gather(d)
            reduce_chunk(d)
            fire(d + 4, d)
            pwrite(d, d)

        def gbody(g, _):
            for d in range(4):
                c = g * 4 + d
                wait_gather(d)
                wait_pwrite(d)
                reduce_chunk(d)
                fire(c + 4, d)
                pwrite(c, d)
            return 0
        lax.fori_loop(1, 31, gbody, 0)

        for d in range(4):
            wait_gather(d)
            wait_pwrite(d)
            reduce_chunk(d)
            pwrite(124 + d, d)
        for d in range(4):
            wait_pwrite(d)

        lane = lax.broadcasted_iota(jnp.int32, (16,), 0)

        def win_body(g, _):
            t0 = g * 16 + lane
            m = jnp.full((16,), -1, jnp.int32)

            def maxj(j, m):
                v = plsc.load_gather(idx_v, [t0 * 16 + j])
                return jnp.maximum(m, v)
            m = lax.fori_loop(0, 16, maxj, m)
            tgtv = m + 1
            iv = wid * 512 + t0
            key = (tgtv << 14) | iv
            skey, _sval = plsc.sort_key_val(key, key)
            tgt_s = skey >> 14
            i_s = skey & 16383
            tscr[...] = tgt_s
            nxt = plsc.load_gather(tscr, [jnp.minimum(lane + 1, 15)])
            is_last = (tgt_s != nxt) | (lane == 15)
            plsc.store_scatter(wloc, [tgt_s], i_s, mask=is_last)
            return 0
        lax.fori_loop(0, 32, win_body, 0)

        pltpu.sync_copy(wloc, shared.at[sid])
        plsc.subcore_barrier()
        pltpu.sync_copy(shared.at[0, pl.ds(sid * 512, 512)], wacc)

        def tile_body(t, _):
            pltpu.sync_copy(shared.at[t, pl.ds(sid * 512, 512)], wtmp)

            def max_body(kk, _):
                wacc[pl.ds(kk * 16, 16)] = jnp.maximum(
                    wacc[pl.ds(kk * 16, 16)], wtmp[pl.ds(kk * 16, 16)])
                return 0
            lax.fori_loop(0, 32, max_body, 0)
            return 0
        lax.fori_loop(1, 16, tile_body, 0)
        pltpu.sync_copy(wacc, wsc_hbm.at[cid, pl.ds(sid * 512, 512)])

    return k(spans1d, E)


def _merge(EC, wsc, nspan):
    nslot = EC.shape[0] - nspan
    mesh = plsc.VectorSubcoreMesh(core_axis_name="c", subcore_axis_name="s")

    @functools.partial(
        pl.kernel, mesh=mesh,
        out_type=jax.ShapeDtypeStruct((nslot, 256), jnp.float32),
        compiler_params=pltpu.CompilerParams(needs_layout_passes=False),
        scratch_types=[
            pltpu.VMEM((256,), jnp.int32),
            pltpu.VMEM((256,), jnp.int32),
            pltpu.VMEM((256,), jnp.int32),
            pltpu.VMEM((2, 128, 256), jnp.float32),
            pltpu.SemaphoreType.DMA,
            pltpu.SemaphoreType.DMA,
            pltpu.SemaphoreType.DMA,
            pltpu.SemaphoreType.DMA,
        ],
    )
    def k(ec_hbm, wsc_hbm, out_hbm, w0, w1, gidx, rows, sg0, sg1, sw0, sw1):
        wid = lax.axis_index("s") * _NC + lax.axis_index("c")
        base = wid * 256
        pltpu.sync_copy(wsc_hbm.at[0, pl.ds(base, 256)], w0)
        pltpu.sync_copy(wsc_hbm.at[1, pl.ds(base, 256)], w1)
        lane = lax.broadcasted_iota(jnp.int32, (16,), 0)

        def idx_body(kk, _):
            sl = pl.ds(kk * 16, 16)
            m = jnp.maximum(w0[sl], w1[sl])
            gidx[sl] = jnp.where(
                m >= 0, m, nspan + base + kk * 16 + lane)
            return 0
        lax.fori_loop(0, 16, idx_body, 0)

        sg = [sg0, sg1]
        sw = [sw0, sw1]
        gcp = {}
        for h in range(2):
            gcp[h] = pltpu.async_copy(
                ec_hbm.at[gidx.at[pl.ds(h * 128, 128)]], rows.at[h], sg[h])
        wcp = {}
        for h in range(2):
            gcp[h].wait()
            wcp[h] = pltpu.async_copy(
                rows.at[h], out_hbm.at[pl.ds(base + h * 128, 128)], sw[h])
        for h in range(2):
            wcp[h].wait()

    return k(EC, wsc)


def _down_body(g_ref, w_ref, b_ref, o_ref):
    o_ref[...] = (
        jnp.dot(g_ref[...], w_ref[...], preferred_element_type=jnp.float32)
        + b_ref[...]
    )


def _down_matmul(g, w, b):
    m, k = g.shape
    n = w.shape[1]
    bm = 512
    return pl.pallas_call(
        _down_body,
        grid=(m // bm,),
        in_specs=[
            pl.BlockSpec((bm, k), lambda i: (i, 0)),
            pl.BlockSpec((k, n), lambda i: (0, 0)),
            pl.BlockSpec((n,), lambda i: (0,)),
        ],
        out_specs=pl.BlockSpec((bm, n), lambda i: (i, 0)),
        out_shape=jax.ShapeDtypeStruct((m, n), jnp.float32),
    )(g, w, b)


def _comp_body(p_ref, w_ref, b_ref, o_ref):
    o_ref[...] = jnp.tanh(
        jnp.dot(p_ref[...], w_ref[...], preferred_element_type=jnp.float32)
        + b_ref[...])


def _comp_matmul(p, w, b):
    m, k = p.shape
    n = w.shape[1]
    bm = 1024
    return pl.pallas_call(
        _comp_body,
        grid=(m // bm,),
        in_specs=[
            pl.BlockSpec((bm, k), lambda i: (i, 0)),
            pl.BlockSpec((k, n), lambda i: (0, 0)),
            pl.BlockSpec((n,), lambda i: (0,)),
        ],
        out_specs=pl.BlockSpec((bm, n), lambda i: (i, 0)),
        out_shape=jax.ShapeDtypeStruct((m, n), jnp.float32),
    )(p, w, b)


def kernel(chunk_input_ids, chunk_tgt_ids, merged_spans,
           opening_nt_start_idx, closing_nt_start_idx,
           emb_table, W_down, b_down, W_comp, b_comp):
    bsz, seqlen = chunk_input_ids.shape
    ids1d = chunk_input_ids.reshape(-1)
    G = _emb_gather(ids1d, emb_table)
    E = _down_matmul(G, W_down, b_down)

    spans1d = merged_spans.reshape(-1)
    pooled, wsc = _pool_and_winner(spans1d, E)
    C = _comp_matmul(pooled, W_comp, b_comp)
    EC = jnp.concatenate([C, E], axis=0)
    out = _merge(EC, wsc, pooled.shape[0])
    return out.reshape(bsz, seqlen, -1)
